# baseline (speedup 1.0000x reference)
import functools

import jax
import jax.numpy as jnp
from jax import lax
from jax.experimental import pallas as pl
from jax.experimental.pallas import tpu as pltpu
from jax.experimental.pallas import tpu_sc as plsc

_VOXEL = (0.2, 0.2)
_PC_MIN = (-25.6, -25.6)
_NX = 256
_NY = 256

_TQ = 512
_TK = 2048

_BIG_I = 2**30


def _three_nn_body(q_ref, k_ref, dist_ref, idx_ref, *, n_known, tk):
    q = q_ref[0]
    k = k_ref[0]
    tq = q.shape[0]
    q3 = q[:, 0:3].astype(jnp.bfloat16)
    qn = q[:, 3:4]

    inf = jnp.float32(jnp.inf)
    v1 = jnp.full((tq, 1), inf, jnp.float32)
    v2 = jnp.full((tq, 1), inf, jnp.float32)
    v3 = jnp.full((tq, 1), inf, jnp.float32)
    i1 = jnp.zeros((tq, 1), jnp.int32)
    i2 = jnp.zeros((tq, 1), jnp.int32)
    i3 = jnp.zeros((tq, 1), jnp.int32)

    n_chunks = n_known // tk
    iota_l = jax.lax.broadcasted_iota(jnp.int32, (tq, tk), 1)
    for c in range(n_chunks):
        kc = k[0:3, c * tk:(c + 1) * tk].astype(jnp.bfloat16)
        knc = k[3:4, c * tk:(c + 1) * tk]
        mm = jax.lax.dot_general(q3, kc, (((1,), (0,)), ((), ())),
                                 preferred_element_type=jnp.float32)
        d = (qn + knc) - 2.0 * mm
        for p in range(3):
            m = jnp.min(d, axis=1, keepdims=True)
            im_l = jnp.min(jnp.where(d == m, iota_l, _BIG_I), axis=1, keepdims=True)
            im = im_l + c * tk
            if p < 2:
                d = jnp.where(iota_l == im_l, inf, d)
            b1 = m < v1
            b2 = m < v2
            b3 = m < v3
            v1n = jnp.where(b1, m, v1)
            i1n = jnp.where(b1, im, i1)
            v2n = jnp.where(b1, v1, jnp.where(b2, m, v2))
            i2n = jnp.where(b1, i1, jnp.where(b2, im, i2))
            v3n = jnp.where(b2, v2, jnp.where(b3, m, v3))
            i3n = jnp.where(b2, i2, jnp.where(b3, im, i3))
            v1, v2, v3, i1, i2, i3 = v1n, v2n, v3n, i1n, i2n, i3n

    dist_ref[0] = jnp.concatenate([v1, v2, v3], axis=1)
    idx_ref[0] = jnp.concatenate([i1, i2, i3], axis=1)


def _three_nn_pallas(q_aug, k_aug, *, interpret=False):
    b, n, _ = q_aug.shape
    m = k_aug.shape[2]
    tq = min(_TQ, n)
    tk = min(_TK, m)
    grid = (b, n // tq)
    return pl.pallas_call(
        functools.partial(_three_nn_body, n_known=m, tk=tk),
        grid=grid,
        in_specs=[
            pl.BlockSpec((1, tq, 4), lambda bi, qi: (bi, qi, 0)),
            pl.BlockSpec((1, 4, m), lambda bi, qi: (bi, 0, 0)),
        ],
        out_specs=[
            pl.BlockSpec((1, tq, 3), lambda bi, qi: (bi, qi, 0)),
            pl.BlockSpec((1, tq, 3), lambda bi, qi: (bi, qi, 0)),
        ],
        out_shape=[
            jax.ShapeDtypeStruct((b, n, 3), jnp.float32),
            jax.ShapeDtypeStruct((b, n, 3), jnp.int32),
        ],
        interpret=interpret,
    )(q_aug, k_aug)


def _sc_scatter(featg_flat, idxs, wb, cells_all):
    ncell = _NY * _NX
    b_sz = cells_all.shape[0]
    nrow = cells_all.shape[2]
    n = nrow * 128

    @functools.partial(
        pl.kernel,
        mesh=plsc.VectorSubcoreMesh(core_axis_name="c", subcore_axis_name="s"),
        compiler_params=pltpu.CompilerParams(use_tc_tiling_on_sc=False),
        out_type=[
            jax.ShapeDtypeStruct((b_sz, 4, ncell, 16), jnp.float32),
            jax.ShapeDtypeStruct((b_sz, 2, ncell, 16), jnp.float32),
        ],
        scratch_types=[
            pltpu.VMEM_SHARED((ncell, 16), jnp.float32),
            pltpu.VMEM((128, 16), jnp.float32),
            pltpu.VMEM((128, 16), jnp.float32),
            pltpu.VMEM((128, 16), jnp.float32),
            pltpu.VMEM((128, 16), jnp.float32),
            pltpu.VMEM((128, 16), jnp.float32),
            pltpu.VMEM((128, 16), jnp.float32),
            pltpu.VMEM((128, 16), jnp.float32),
            pltpu.VMEM((16, 128), jnp.int32),
            pltpu.VMEM((16, 128), jnp.int32),
            pltpu.VMEM((16, 128), jnp.int32),
            pltpu.VMEM((16, 128), jnp.int32),
            pltpu.VMEM((1024, 16), jnp.float32),
            pltpu.VMEM((128, 16), jnp.float32),
            pltpu.SemaphoreType.DMA,
        ],
    )
    def body(featg_hbm, idx_hbm, wb_hbm, cells_hbm, sums_hbm, cnt_hbm,
             slab, fbuf, r1, r2, w0, w1, w2, obuf, cbuf, i0, i1, i2,
             zbuf, onesb, sem):
        c = lax.axis_index("c")
        s = lax.axis_index("s")

        def fb(i, _):
            zbuf[i] = jnp.zeros((16,), jnp.float32)
            return 0
        lax.fori_loop(0, 1024, fb, 0)

        def ob(i, _):
            onesb[i] = jnp.ones((16,), jnp.float32)
            return 0
        lax.fori_loop(0, 128, ob, 0)

        def zero_slab():
            for i in range(4):
                pltpu.sync_copy(zbuf, slab.at[pl.ds((s * 4 + i) * 1024, 1024)])

        def copy_out(dst):
            pltpu.sync_copy(slab.at[pl.ds(s * 4096, 4096)],
                            dst.at[pl.ds(s * 4096, 4096)])

        for b in range(b_sz):
            for gl in range(2):
                g = c * 2 + gl
                zero_slab()
                plsc.subcore_barrier()
                pltpu.sync_copy(cells_hbm.at[b, 0, pl.ds(s * 16, 16)], cbuf)
                base_flat = (b * 4 + g) * n + s * 2048

                def known_j(j, _):
                    pltpu.sync_copy(
                        featg_hbm.at[pl.ds(base_flat + j * 128, 128)], fbuf)
                    pltpu.sync_copy(fbuf, slab.at[cbuf.at[j]], add=True)
                    return 0
                lax.fori_loop(0, 16, known_j, 0)

                pltpu.sync_copy(cells_hbm.at[b, 1, pl.ds(s * 16, 16)], cbuf)
                pltpu.sync_copy(idx_hbm.at[b, g, 0, pl.ds(s * 16, 16)], i0)
                pltpu.sync_copy(idx_hbm.at[b, g, 1, pl.ds(s * 16, 16)], i1)
                pltpu.sync_copy(idx_hbm.at[b, g, 2, pl.ds(s * 16, 16)], i2)

                def far_j(j, _):
                    pbase = s * 2048 + j * 128
                    a0 = pltpu.async_copy(featg_hbm.at[i0.at[j]], fbuf, sem)
                    a1 = pltpu.async_copy(featg_hbm.at[i1.at[j]], r1, sem)
                    a2 = pltpu.async_copy(featg_hbm.at[i2.at[j]], r2, sem)
                    pltpu.sync_copy(wb_hbm.at[b, 0, pl.ds(pbase, 128)], w0)
                    pltpu.sync_copy(wb_hbm.at[b, 1, pl.ds(pbase, 128)], w1)
                    pltpu.sync_copy(wb_hbm.at[b, 2, pl.ds(pbase, 128)], w2)
                    a0.wait()
                    a1.wait()
                    a2.wait()

                    def interp(p, _):
                        obuf[p] = (fbuf[p] * w0[p] + r1[p] * w1[p]
                                   + r2[p] * w2[p])
                        return 0
                    lax.fori_loop(0, 128, interp, 0)
                    pltpu.sync_copy(obuf, slab.at[cbuf.at[j]], add=True)
                    return 0
                lax.fori_loop(0, 16, far_j, 0)
                plsc.subcore_barrier()
                copy_out(sums_hbm.at[b, g])
                plsc.subcore_barrier()

            zero_slab()
            plsc.subcore_barrier()
            pltpu.sync_copy(cells_hbm.at[b, c, pl.ds(s * 16, 16)], cbuf)

            def cnt_j(j, _):
                pltpu.sync_copy(onesb, slab.at[cbuf.at[j]], add=True)
                return 0
            lax.fori_loop(0, 16, cnt_j, 0)
            plsc.subcore_barrier()
            copy_out(cnt_hbm.at[b, c])
            plsc.subcore_barrier()

    return body(featg_flat, idxs, wb, cells_all)


def _cells(points):
    xi = jnp.clip(jnp.floor((points[..., 0] - _PC_MIN[0]) / _VOXEL[0]).astype(jnp.int32), 0, _NX - 1)
    yi = jnp.clip(jnp.floor((points[..., 1] - _PC_MIN[1]) / _VOXEL[1]).astype(jnp.int32), 0, _NY - 1)
    return yi * _NX + xi


def kernel(fv_features, points_img, proj_masks, points_img_far, proj_masks_far):
    b, c, h, w = fv_features.shape
    n = h * w
    feats = jnp.transpose(fv_features, (0, 2, 3, 1)).reshape(b, n, c)
    pts = jnp.transpose(points_img[:, :3], (0, 2, 3, 1)).reshape(b, n, 3)
    pts_far = jnp.transpose(points_img_far[:, :3], (0, 2, 3, 1)).reshape(b, n, 3)

    qn = (pts_far ** 2).sum(-1)
    kn = (pts ** 2).sum(-1)
    q_aug = jnp.concatenate([pts_far, qn[..., None]], axis=-1)
    k_aug = jnp.concatenate(
        [jnp.transpose(pts, (0, 2, 1)), kn[:, None, :]], axis=1)

    dist, idx = _three_nn_pallas(q_aug, k_aug)

    dist = jnp.maximum(dist, 0.0)
    recip = 1.0 / (dist + 1e-8)
    weight = recip / recip.sum(axis=-1, keepdims=True)

    cells_known = _cells(pts)
    cells_far = _cells(pts_far)

    featg_flat = jnp.transpose(feats.reshape(b, n, 4, 16),
                               (0, 2, 1, 3)).reshape(b * 4 * n, 16)
    offs = (jnp.arange(b, dtype=jnp.int32)[:, None] * 4
            + jnp.arange(4, dtype=jnp.int32)[None, :]) * n
    idx_t = jnp.transpose(idx, (0, 2, 1))
    idxs = (idx_t[:, None] + offs[:, :, None, None]).reshape(
        b, 4, 3, n // 128, 128)
    wb = jnp.broadcast_to(
        jnp.transpose(weight, (0, 2, 1))[..., None], (b, 3, n, 16))
    cells_all = jnp.stack([cells_known, cells_far], axis=1).reshape(
        b, 2, n // 128, 128)

    sums, cnt_part = _sc_scatter(featg_flat, idxs, wb, cells_all)

    cnt = cnt_part[:, 0, :, 0] + cnt_part[:, 1, :, 0]
    sums64 = jnp.transpose(sums, (0, 1, 3, 2)).reshape(b, c, _NY * _NX)
    bev = sums64 / jnp.maximum(cnt, 1.0)[:, None]
    return bev.reshape(b, c, _NY, _NX)


# speedup vs baseline: 1.1958x; 1.1958x over previous
import functools

import jax
import jax.numpy as jnp
from jax import lax
from jax.experimental import pallas as pl
from jax.experimental.pallas import tpu as pltpu
from jax.experimental.pallas import tpu_sc as plsc

_VOXEL = (0.2, 0.2)
_PC_MIN = (-25.6, -25.6)
_NX = 256
_NY = 256

_TQ = 256
_TK = 4096

_BIG_I = 2**30


def _three_nn_body(q_ref, k_ref, dist_ref, idx_ref, *, n_known, tk):
    q = q_ref[0]
    k = k_ref[0]
    tq = q.shape[0]
    q3 = q[:, 0:3].astype(jnp.bfloat16)
    qn = q[:, 3:4]

    inf = jnp.float32(jnp.inf)
    v1 = jnp.full((tq, 1), inf, jnp.float32)
    v2 = jnp.full((tq, 1), inf, jnp.float32)
    v3 = jnp.full((tq, 1), inf, jnp.float32)
    i1 = jnp.zeros((tq, 1), jnp.int32)
    i2 = jnp.zeros((tq, 1), jnp.int32)
    i3 = jnp.zeros((tq, 1), jnp.int32)

    n_chunks = n_known // tk
    iota_l = jax.lax.broadcasted_iota(jnp.int32, (tq, tk), 1)
    for c in range(n_chunks):
        kc = k[0:3, c * tk:(c + 1) * tk].astype(jnp.bfloat16)
        knc = k[3:4, c * tk:(c + 1) * tk]
        mm = jax.lax.dot_general(q3, kc, (((1,), (0,)), ((), ())),
                                 preferred_element_type=jnp.float32)
        d = (qn + knc) - 2.0 * mm
        for p in range(3):
            m = jnp.min(d, axis=1, keepdims=True)
            im_l = jnp.min(jnp.where(d == m, iota_l, _BIG_I), axis=1, keepdims=True)
            im = im_l + c * tk
            if p < 2:
                d = jnp.where(iota_l == im_l, inf, d)
            b1 = m < v1
            b2 = m < v2
            b3 = m < v3
            v1n = jnp.where(b1, m, v1)
            i1n = jnp.where(b1, im, i1)
            v2n = jnp.where(b1, v1, jnp.where(b2, m, v2))
            i2n = jnp.where(b1, i1, jnp.where(b2, im, i2))
            v3n = jnp.where(b2, v2, jnp.where(b3, m, v3))
            i3n = jnp.where(b2, i2, jnp.where(b3, im, i3))
            v1, v2, v3, i1, i2, i3 = v1n, v2n, v3n, i1n, i2n, i3n

    dist_ref[0] = jnp.concatenate([v1, v2, v3], axis=1)
    idx_ref[0] = jnp.concatenate([i1, i2, i3], axis=1)


def _three_nn_pallas(q_aug, k_aug, *, interpret=False):
    b, n, _ = q_aug.shape
    m = k_aug.shape[2]
    tq = min(_TQ, n)
    tk = min(_TK, m)
    grid = (b, n // tq)
    return pl.pallas_call(
        functools.partial(_three_nn_body, n_known=m, tk=tk),
        grid=grid,
        in_specs=[
            pl.BlockSpec((1, tq, 4), lambda bi, qi: (bi, qi, 0)),
            pl.BlockSpec((1, 4, m), lambda bi, qi: (bi, 0, 0)),
        ],
        out_specs=[
            pl.BlockSpec((1, tq, 3), lambda bi, qi: (bi, qi, 0)),
            pl.BlockSpec((1, tq, 3), lambda bi, qi: (bi, qi, 0)),
        ],
        out_shape=[
            jax.ShapeDtypeStruct((b, n, 3), jnp.float32),
            jax.ShapeDtypeStruct((b, n, 3), jnp.int32),
        ],
        interpret=interpret,
    )(q_aug, k_aug)


def _sc_scatter(featg_flat, idxs, wb, cells_all):
    ncell = _NY * _NX
    b_sz = cells_all.shape[0]
    nrow = cells_all.shape[2]
    n = nrow * 128

    @functools.partial(
        pl.kernel,
        mesh=plsc.VectorSubcoreMesh(core_axis_name="c", subcore_axis_name="s"),
        compiler_params=pltpu.CompilerParams(use_tc_tiling_on_sc=False),
        out_type=[
            jax.ShapeDtypeStruct((b_sz, 4, ncell, 16), jnp.float32),
            jax.ShapeDtypeStruct((b_sz, 2, ncell, 16), jnp.float32),
        ],
        scratch_types=[
            pltpu.VMEM_SHARED((ncell, 16), jnp.float32),
            pltpu.VMEM((128, 16), jnp.float32),
            pltpu.VMEM((128, 16), jnp.float32),
            pltpu.VMEM((128, 16), jnp.float32),
            pltpu.VMEM((128, 16), jnp.float32),
            pltpu.VMEM((128, 16), jnp.float32),
            pltpu.VMEM((128, 16), jnp.float32),
            pltpu.VMEM((128, 16), jnp.float32),
            pltpu.VMEM((16, 128), jnp.int32),
            pltpu.VMEM((16, 128), jnp.int32),
            pltpu.VMEM((16, 128), jnp.int32),
            pltpu.VMEM((16, 128), jnp.int32),
            pltpu.VMEM((1024, 16), jnp.float32),
            pltpu.VMEM((128, 16), jnp.float32),
            pltpu.SemaphoreType.DMA,
        ],
    )
    def body(featg_hbm, idx_hbm, wb_hbm, cells_hbm, sums_hbm, cnt_hbm,
             slab, fbuf, r1, r2, w0, w1, w2, obuf, cbuf, i0, i1, i2,
             zbuf, onesb, sem):
        c = lax.axis_index("c")
        s = lax.axis_index("s")

        def fb(i, _):
            zbuf[i] = jnp.zeros((16,), jnp.float32)
            return 0
        lax.fori_loop(0, 1024, fb, 0)

        def ob(i, _):
            onesb[i] = jnp.ones((16,), jnp.float32)
            return 0
        lax.fori_loop(0, 128, ob, 0)

        def zero_slab():
            for i in range(4):
                pltpu.sync_copy(zbuf, slab.at[pl.ds((s * 4 + i) * 1024, 1024)])

        def copy_out(dst):
            pltpu.sync_copy(slab.at[pl.ds(s * 4096, 4096)],
                            dst.at[pl.ds(s * 4096, 4096)])

        for b in range(b_sz):
            for gl in range(2):
                g = c * 2 + gl
                zero_slab()
                plsc.subcore_barrier()
                pltpu.sync_copy(cells_hbm.at[b, 0, pl.ds(s * 16, 16)], cbuf)
                base_flat = (b * 4 + g) * n + s * 2048

                def known_j(j, _):
                    pltpu.sync_copy(
                        featg_hbm.at[pl.ds(base_flat + j * 128, 128)], fbuf)
                    pltpu.sync_copy(fbuf, slab.at[cbuf.at[j]], add=True)
                    return 0
                lax.fori_loop(0, 16, known_j, 0)

                pltpu.sync_copy(cells_hbm.at[b, 1, pl.ds(s * 16, 16)], cbuf)
                pltpu.sync_copy(idx_hbm.at[b, g, 0, pl.ds(s * 16, 16)], i0)
                pltpu.sync_copy(idx_hbm.at[b, g, 1, pl.ds(s * 16, 16)], i1)
                pltpu.sync_copy(idx_hbm.at[b, g, 2, pl.ds(s * 16, 16)], i2)

                def far_j(j, _):
                    pbase = s * 2048 + j * 128
                    a0 = pltpu.async_copy(featg_hbm.at[i0.at[j]], fbuf, sem)
                    a1 = pltpu.async_copy(featg_hbm.at[i1.at[j]], r1, sem)
                    a2 = pltpu.async_copy(featg_hbm.at[i2.at[j]], r2, sem)
                    pltpu.sync_copy(wb_hbm.at[b, 0, pl.ds(pbase, 128)], w0)
                    pltpu.sync_copy(wb_hbm.at[b, 1, pl.ds(pbase, 128)], w1)
                    pltpu.sync_copy(wb_hbm.at[b, 2, pl.ds(pbase, 128)], w2)
                    a0.wait()
                    a1.wait()
                    a2.wait()

                    def interp(p, _):
                        obuf[p] = (fbuf[p] * w0[p] + r1[p] * w1[p]
                                   + r2[p] * w2[p])
                        return 0
                    lax.fori_loop(0, 128, interp, 0)
                    pltpu.sync_copy(obuf, slab.at[cbuf.at[j]], add=True)
                    return 0
                lax.fori_loop(0, 16, far_j, 0)
                plsc.subcore_barrier()
                copy_out(sums_hbm.at[b, g])
                plsc.subcore_barrier()

            zero_slab()
            plsc.subcore_barrier()
            pltpu.sync_copy(cells_hbm.at[b, c, pl.ds(s * 16, 16)], cbuf)

            def cnt_j(j, _):
                pltpu.sync_copy(onesb, slab.at[cbuf.at[j]], add=True)
                return 0
            lax.fori_loop(0, 16, cnt_j, 0)
            plsc.subcore_barrier()
            copy_out(cnt_hbm.at[b, c])
            plsc.subcore_barrier()

    return body(featg_flat, idxs, wb, cells_all)


def _cells(points):
    xi = jnp.clip(jnp.floor((points[..., 0] - _PC_MIN[0]) / _VOXEL[0]).astype(jnp.int32), 0, _NX - 1)
    yi = jnp.clip(jnp.floor((points[..., 1] - _PC_MIN[1]) / _VOXEL[1]).astype(jnp.int32), 0, _NY - 1)
    return yi * _NX + xi


def kernel(fv_features, points_img, proj_masks, points_img_far, proj_masks_far):
    b, c, h, w = fv_features.shape
    n = h * w
    feats = jnp.transpose(fv_features, (0, 2, 3, 1)).reshape(b, n, c)
    pts = jnp.transpose(points_img[:, :3], (0, 2, 3, 1)).reshape(b, n, 3)
    pts_far = jnp.transpose(points_img_far[:, :3], (0, 2, 3, 1)).reshape(b, n, 3)

    qn = (pts_far ** 2).sum(-1)
    kn = (pts ** 2).sum(-1)
    q_aug = jnp.concatenate([pts_far, qn[..., None]], axis=-1)
    k_aug = jnp.concatenate(
        [jnp.transpose(pts, (0, 2, 1)), kn[:, None, :]], axis=1)

    dist, idx = _three_nn_pallas(q_aug, k_aug)

    dist = jnp.maximum(dist, 0.0)
    recip = 1.0 / (dist + 1e-8)
    weight = recip / recip.sum(axis=-1, keepdims=True)

    cells_known = _cells(pts)
    cells_far = _cells(pts_far)

    featg_flat = jnp.transpose(feats.reshape(b, n, 4, 16),
                               (0, 2, 1, 3)).reshape(b * 4 * n, 16)
    offs = (jnp.arange(b, dtype=jnp.int32)[:, None] * 4
            + jnp.arange(4, dtype=jnp.int32)[None, :]) * n
    idx_t = jnp.transpose(idx, (0, 2, 1))
    idxs = (idx_t[:, None] + offs[:, :, None, None]).reshape(
        b, 4, 3, n // 128, 128)
    wb = jnp.broadcast_to(
        jnp.transpose(weight, (0, 2, 1))[..., None], (b, 3, n, 16))
    cells_all = jnp.stack([cells_known, cells_far], axis=1).reshape(
        b, 2, n // 128, 128)

    sums, cnt_part = _sc_scatter(featg_flat, idxs, wb, cells_all)

    cnt = cnt_part[:, 0, :, 0] + cnt_part[:, 1, :, 0]
    sums64 = jnp.transpose(sums, (0, 1, 3, 2)).reshape(b, c, _NY * _NX)
    bev = sums64 / jnp.maximum(cnt, 1.0)[:, None]
    return bev.reshape(b, c, _NY, _NX)


# speedup vs baseline: 1.2486x; 1.0442x over previous
import functools

import jax
import jax.numpy as jnp
from jax import lax
from jax.experimental import pallas as pl
from jax.experimental.pallas import tpu as pltpu
from jax.experimental.pallas import tpu_sc as plsc

_VOXEL = (0.2, 0.2)
_PC_MIN = (-25.6, -25.6)
_NX = 256
_NY = 256

_TQ = 256
_TK = 8192

_BIG_I = 2**30


def _three_nn_body(q_ref, k_ref, dist_ref, idx_ref, *, n_known, tk):
    q = q_ref[0]
    k = k_ref[0]
    tq = q.shape[0]
    q3 = q[:, 0:3].astype(jnp.bfloat16)
    qn = q[:, 3:4]

    inf = jnp.float32(jnp.inf)
    v1 = jnp.full((tq, 1), inf, jnp.float32)
    v2 = jnp.full((tq, 1), inf, jnp.float32)
    v3 = jnp.full((tq, 1), inf, jnp.float32)
    i1 = jnp.zeros((tq, 1), jnp.int32)
    i2 = jnp.zeros((tq, 1), jnp.int32)
    i3 = jnp.zeros((tq, 1), jnp.int32)

    n_chunks = n_known // tk
    iota_l = jax.lax.broadcasted_iota(jnp.int32, (tq, tk), 1)
    for c in range(n_chunks):
        kc = k[0:3, c * tk:(c + 1) * tk].astype(jnp.bfloat16)
        knc = k[3:4, c * tk:(c + 1) * tk]
        mm = jax.lax.dot_general(q3, kc, (((1,), (0,)), ((), ())),
                                 preferred_element_type=jnp.float32)
        d = (qn + knc) - 2.0 * mm
        for p in range(3):
            m = jnp.min(d, axis=1, keepdims=True)
            im_l = jnp.min(jnp.where(d == m, iota_l, _BIG_I), axis=1, keepdims=True)
            im = im_l + c * tk
            if p < 2:
                d = jnp.where(iota_l == im_l, inf, d)
            b1 = m < v1
            b2 = m < v2
            b3 = m < v3
            v1n = jnp.where(b1, m, v1)
            i1n = jnp.where(b1, im, i1)
            v2n = jnp.where(b1, v1, jnp.where(b2, m, v2))
            i2n = jnp.where(b1, i1, jnp.where(b2, im, i2))
            v3n = jnp.where(b2, v2, jnp.where(b3, m, v3))
            i3n = jnp.where(b2, i2, jnp.where(b3, im, i3))
            v1, v2, v3, i1, i2, i3 = v1n, v2n, v3n, i1n, i2n, i3n

    dist_ref[0] = jnp.concatenate([v1, v2, v3], axis=1)
    idx_ref[0] = jnp.concatenate([i1, i2, i3], axis=1)


def _three_nn_pallas(q_aug, k_aug, *, interpret=False):
    b, n, _ = q_aug.shape
    m = k_aug.shape[2]
    tq = min(_TQ, n)
    tk = min(_TK, m)
    grid = (b, n // tq)
    return pl.pallas_call(
        functools.partial(_three_nn_body, n_known=m, tk=tk),
        grid=grid,
        in_specs=[
            pl.BlockSpec((1, tq, 4), lambda bi, qi: (bi, qi, 0)),
            pl.BlockSpec((1, 4, m), lambda bi, qi: (bi, 0, 0)),
        ],
        out_specs=[
            pl.BlockSpec((1, tq, 3), lambda bi, qi: (bi, qi, 0)),
            pl.BlockSpec((1, tq, 3), lambda bi, qi: (bi, qi, 0)),
        ],
        out_shape=[
            jax.ShapeDtypeStruct((b, n, 3), jnp.float32),
            jax.ShapeDtypeStruct((b, n, 3), jnp.int32),
        ],
        interpret=interpret,
    )(q_aug, k_aug)


def _sc_scatter(featg_flat, idxs, wb, cells_all):
    ncell = _NY * _NX
    b_sz = cells_all.shape[0]
    nrow = cells_all.shape[2]
    n = nrow * 128

    @functools.partial(
        pl.kernel,
        mesh=plsc.VectorSubcoreMesh(core_axis_name="c", subcore_axis_name="s"),
        compiler_params=pltpu.CompilerParams(use_tc_tiling_on_sc=False),
        out_type=[
            jax.ShapeDtypeStruct((b_sz, 4, ncell, 16), jnp.float32),
            jax.ShapeDtypeStruct((b_sz, 2, ncell, 16), jnp.float32),
        ],
        scratch_types=[
            pltpu.VMEM_SHARED((ncell, 16), jnp.float32),
            pltpu.VMEM((128, 16), jnp.float32),
            pltpu.VMEM((128, 16), jnp.float32),
            pltpu.VMEM((128, 16), jnp.float32),
            pltpu.VMEM((128, 16), jnp.float32),
            pltpu.VMEM((128, 16), jnp.float32),
            pltpu.VMEM((128, 16), jnp.float32),
            pltpu.VMEM((128, 16), jnp.float32),
            pltpu.VMEM((16, 128), jnp.int32),
            pltpu.VMEM((16, 128), jnp.int32),
            pltpu.VMEM((16, 128), jnp.int32),
            pltpu.VMEM((16, 128), jnp.int32),
            pltpu.VMEM((1024, 16), jnp.float32),
            pltpu.VMEM((128, 16), jnp.float32),
            pltpu.SemaphoreType.DMA,
        ],
    )
    def body(featg_hbm, idx_hbm, wb_hbm, cells_hbm, sums_hbm, cnt_hbm,
             slab, fbuf, r1, r2, w0, w1, w2, obuf, cbuf, i0, i1, i2,
             zbuf, onesb, sem):
        c = lax.axis_index("c")
        s = lax.axis_index("s")

        def fb(i, _):
            zbuf[i] = jnp.zeros((16,), jnp.float32)
            return 0
        lax.fori_loop(0, 1024, fb, 0)

        def ob(i, _):
            onesb[i] = jnp.ones((16,), jnp.float32)
            return 0
        lax.fori_loop(0, 128, ob, 0)

        def zero_slab():
            for i in range(4):
                pltpu.sync_copy(zbuf, slab.at[pl.ds((s * 4 + i) * 1024, 1024)])

        def copy_out(dst):
            pltpu.sync_copy(slab.at[pl.ds(s * 4096, 4096)],
                            dst.at[pl.ds(s * 4096, 4096)])

        for b in range(b_sz):
            for gl in range(2):
                g = c * 2 + gl
                zero_slab()
                plsc.subcore_barrier()
                pltpu.sync_copy(cells_hbm.at[b, 0, pl.ds(s * 16, 16)], cbuf)
                base_flat = (b * 4 + g) * n + s * 2048

                def known_j(j, _):
                    pltpu.sync_copy(
                        featg_hbm.at[pl.ds(base_flat + j * 128, 128)], fbuf)
                    pltpu.sync_copy(fbuf, slab.at[cbuf.at[j]], add=True)
                    return 0
                lax.fori_loop(0, 16, known_j, 0)

                pltpu.sync_copy(cells_hbm.at[b, 1, pl.ds(s * 16, 16)], cbuf)
                pltpu.sync_copy(idx_hbm.at[b, g, 0, pl.ds(s * 16, 16)], i0)
                pltpu.sync_copy(idx_hbm.at[b, g, 1, pl.ds(s * 16, 16)], i1)
                pltpu.sync_copy(idx_hbm.at[b, g, 2, pl.ds(s * 16, 16)], i2)

                def far_j(j, _):
                    pbase = s * 2048 + j * 128
                    a0 = pltpu.async_copy(featg_hbm.at[i0.at[j]], fbuf, sem)
                    a1 = pltpu.async_copy(featg_hbm.at[i1.at[j]], r1, sem)
                    a2 = pltpu.async_copy(featg_hbm.at[i2.at[j]], r2, sem)
                    pltpu.sync_copy(wb_hbm.at[b, 0, pl.ds(pbase, 128)], w0)
                    pltpu.sync_copy(wb_hbm.at[b, 1, pl.ds(pbase, 128)], w1)
                    pltpu.sync_copy(wb_hbm.at[b, 2, pl.ds(pbase, 128)], w2)
                    a0.wait()
                    a1.wait()
                    a2.wait()

                    def interp(p, _):
                        obuf[p] = (fbuf[p] * w0[p] + r1[p] * w1[p]
                                   + r2[p] * w2[p])
                        return 0
                    lax.fori_loop(0, 128, interp, 0)
                    pltpu.sync_copy(obuf, slab.at[cbuf.at[j]], add=True)
                    return 0
                lax.fori_loop(0, 16, far_j, 0)
                plsc.subcore_barrier()
                copy_out(sums_hbm.at[b, g])
                plsc.subcore_barrier()

            zero_slab()
            plsc.subcore_barrier()
            pltpu.sync_copy(cells_hbm.at[b, c, pl.ds(s * 16, 16)], cbuf)

            def cnt_j(j, _):
                pltpu.sync_copy(onesb, slab.at[cbuf.at[j]], add=True)
                return 0
            lax.fori_loop(0, 16, cnt_j, 0)
            plsc.subcore_barrier()
            copy_out(cnt_hbm.at[b, c])
            plsc.subcore_barrier()

    return body(featg_flat, idxs, wb, cells_all)


def _cells(points):
    xi = jnp.clip(jnp.floor((points[..., 0] - _PC_MIN[0]) / _VOXEL[0]).astype(jnp.int32), 0, _NX - 1)
    yi = jnp.clip(jnp.floor((points[..., 1] - _PC_MIN[1]) / _VOXEL[1]).astype(jnp.int32), 0, _NY - 1)
    return yi * _NX + xi


def kernel(fv_features, points_img, proj_masks, points_img_far, proj_masks_far):
    b, c, h, w = fv_features.shape
    n = h * w
    feats = jnp.transpose(fv_features, (0, 2, 3, 1)).reshape(b, n, c)
    pts = jnp.transpose(points_img[:, :3], (0, 2, 3, 1)).reshape(b, n, 3)
    pts_far = jnp.transpose(points_img_far[:, :3], (0, 2, 3, 1)).reshape(b, n, 3)

    qn = (pts_far ** 2).sum(-1)
    kn = (pts ** 2).sum(-1)
    q_aug = jnp.concatenate([pts_far, qn[..., None]], axis=-1)
    k_aug = jnp.concatenate(
        [jnp.transpose(pts, (0, 2, 1)), kn[:, None, :]], axis=1)

    dist, idx = _three_nn_pallas(q_aug, k_aug)

    dist = jnp.maximum(dist, 0.0)
    recip = 1.0 / (dist + 1e-8)
    weight = recip / recip.sum(axis=-1, keepdims=True)

    cells_known = _cells(pts)
    cells_far = _cells(pts_far)

    featg_flat = jnp.transpose(feats.reshape(b, n, 4, 16),
                               (0, 2, 1, 3)).reshape(b * 4 * n, 16)
    offs = (jnp.arange(b, dtype=jnp.int32)[:, None] * 4
            + jnp.arange(4, dtype=jnp.int32)[None, :]) * n
    idx_t = jnp.transpose(idx, (0, 2, 1))
    idxs = (idx_t[:, None] + offs[:, :, None, None]).reshape(
        b, 4, 3, n // 128, 128)
    wb = jnp.broadcast_to(
        jnp.transpose(weight, (0, 2, 1))[..., None], (b, 3, n, 16))
    cells_all = jnp.stack([cells_known, cells_far], axis=1).reshape(
        b, 2, n // 128, 128)

    sums, cnt_part = _sc_scatter(featg_flat, idxs, wb, cells_all)

    cnt = cnt_part[:, 0, :, 0] + cnt_part[:, 1, :, 0]
    sums64 = jnp.transpose(sums, (0, 1, 3, 2)).reshape(b, c, _NY * _NX)
    bev = sums64 / jnp.maximum(cnt, 1.0)[:, None]
    return bev.reshape(b, c, _NY, _NX)


# speedup vs baseline: 1.2556x; 1.0056x over previous
import functools

import jax
import jax.numpy as jnp
from jax import lax
from jax.experimental import pallas as pl
from jax.experimental.pallas import tpu as pltpu
from jax.experimental.pallas import tpu_sc as plsc

_VOXEL = (0.2, 0.2)
_PC_MIN = (-25.6, -25.6)
_NX = 256
_NY = 256

_TQ = 256
_TK = 16384

_BIG_I = 2**30


def _three_nn_body(q_ref, k_ref, dist_ref, idx_ref, *, n_known, tk):
    q = q_ref[0]
    k = k_ref[0]
    tq = q.shape[0]
    q3 = q[:, 0:3].astype(jnp.bfloat16)
    qn = q[:, 3:4]

    inf = jnp.float32(jnp.inf)
    v1 = jnp.full((tq, 1), inf, jnp.float32)
    v2 = jnp.full((tq, 1), inf, jnp.float32)
    v3 = jnp.full((tq, 1), inf, jnp.float32)
    i1 = jnp.zeros((tq, 1), jnp.int32)
    i2 = jnp.zeros((tq, 1), jnp.int32)
    i3 = jnp.zeros((tq, 1), jnp.int32)

    n_chunks = n_known // tk
    iota_l = jax.lax.broadcasted_iota(jnp.int32, (tq, tk), 1)
    for c in range(n_chunks):
        kc = k[0:3, c * tk:(c + 1) * tk].astype(jnp.bfloat16)
        knc = k[3:4, c * tk:(c + 1) * tk]
        mm = jax.lax.dot_general(q3, kc, (((1,), (0,)), ((), ())),
                                 preferred_element_type=jnp.float32)
        d = (qn + knc) - 2.0 * mm
        for p in range(3):
            m = jnp.min(d, axis=1, keepdims=True)
            im_l = jnp.min(jnp.where(d == m, iota_l, _BIG_I), axis=1, keepdims=True)
            im = im_l + c * tk
            if p < 2:
                d = jnp.where(iota_l == im_l, inf, d)
            b1 = m < v1
            b2 = m < v2
            b3 = m < v3
            v1n = jnp.where(b1, m, v1)
            i1n = jnp.where(b1, im, i1)
            v2n = jnp.where(b1, v1, jnp.where(b2, m, v2))
            i2n = jnp.where(b1, i1, jnp.where(b2, im, i2))
            v3n = jnp.where(b2, v2, jnp.where(b3, m, v3))
            i3n = jnp.where(b2, i2, jnp.where(b3, im, i3))
            v1, v2, v3, i1, i2, i3 = v1n, v2n, v3n, i1n, i2n, i3n

    dist_ref[0] = jnp.concatenate([v1, v2, v3], axis=1)
    idx_ref[0] = jnp.concatenate([i1, i2, i3], axis=1)


def _three_nn_pallas(q_aug, k_aug, *, interpret=False):
    b, n, _ = q_aug.shape
    m = k_aug.shape[2]
    tq = min(_TQ, n)
    tk = min(_TK, m)
    grid = (b, n // tq)
    return pl.pallas_call(
        functools.partial(_three_nn_body, n_known=m, tk=tk),
        grid=grid,
        in_specs=[
            pl.BlockSpec((1, tq, 4), lambda bi, qi: (bi, qi, 0)),
            pl.BlockSpec((1, 4, m), lambda bi, qi: (bi, 0, 0)),
        ],
        out_specs=[
            pl.BlockSpec((1, tq, 3), lambda bi, qi: (bi, qi, 0)),
            pl.BlockSpec((1, tq, 3), lambda bi, qi: (bi, qi, 0)),
        ],
        out_shape=[
            jax.ShapeDtypeStruct((b, n, 3), jnp.float32),
            jax.ShapeDtypeStruct((b, n, 3), jnp.int32),
        ],
        interpret=interpret,
    )(q_aug, k_aug)


def _sc_scatter(featg_flat, idxs, wb, cells_all):
    ncell = _NY * _NX
    b_sz = cells_all.shape[0]
    nrow = cells_all.shape[2]
    n = nrow * 128

    @functools.partial(
        pl.kernel,
        mesh=plsc.VectorSubcoreMesh(core_axis_name="c", subcore_axis_name="s"),
        compiler_params=pltpu.CompilerParams(use_tc_tiling_on_sc=False),
        out_type=[
            jax.ShapeDtypeStruct((b_sz, 4, ncell, 16), jnp.float32),
            jax.ShapeDtypeStruct((b_sz, 2, ncell, 16), jnp.float32),
        ],
        scratch_types=[
            pltpu.VMEM_SHARED((ncell, 16), jnp.float32),
            pltpu.VMEM((128, 16), jnp.float32),
            pltpu.VMEM((128, 16), jnp.float32),
            pltpu.VMEM((128, 16), jnp.float32),
            pltpu.VMEM((128, 16), jnp.float32),
            pltpu.VMEM((128, 16), jnp.float32),
            pltpu.VMEM((128, 16), jnp.float32),
            pltpu.VMEM((128, 16), jnp.float32),
            pltpu.VMEM((16, 128), jnp.int32),
            pltpu.VMEM((16, 128), jnp.int32),
            pltpu.VMEM((16, 128), jnp.int32),
            pltpu.VMEM((16, 128), jnp.int32),
            pltpu.VMEM((1024, 16), jnp.float32),
            pltpu.VMEM((128, 16), jnp.float32),
            pltpu.SemaphoreType.DMA,
        ],
    )
    def body(featg_hbm, idx_hbm, wb_hbm, cells_hbm, sums_hbm, cnt_hbm,
             slab, fbuf, r1, r2, w0, w1, w2, obuf, cbuf, i0, i1, i2,
             zbuf, onesb, sem):
        c = lax.axis_index("c")
        s = lax.axis_index("s")

        def fb(i, _):
            zbuf[i] = jnp.zeros((16,), jnp.float32)
            return 0
        lax.fori_loop(0, 1024, fb, 0)

        def ob(i, _):
            onesb[i] = jnp.ones((16,), jnp.float32)
            return 0
        lax.fori_loop(0, 128, ob, 0)

        def zero_slab():
            for i in range(4):
                pltpu.sync_copy(zbuf, slab.at[pl.ds((s * 4 + i) * 1024, 1024)])

        def copy_out(dst):
            pltpu.sync_copy(slab.at[pl.ds(s * 4096, 4096)],
                            dst.at[pl.ds(s * 4096, 4096)])

        for b in range(b_sz):
            for gl in range(2):
                g = c * 2 + gl
                zero_slab()
                plsc.subcore_barrier()
                pltpu.sync_copy(cells_hbm.at[b, 0, pl.ds(s * 16, 16)], cbuf)
                base_flat = (b * 4 + g) * n + s * 2048

                def known_j(j, _):
                    pltpu.sync_copy(
                        featg_hbm.at[pl.ds(base_flat + j * 128, 128)], fbuf)
                    pltpu.sync_copy(fbuf, slab.at[cbuf.at[j]], add=True)
                    return 0
                lax.fori_loop(0, 16, known_j, 0)

                pltpu.sync_copy(cells_hbm.at[b, 1, pl.ds(s * 16, 16)], cbuf)
                pltpu.sync_copy(idx_hbm.at[b, g, 0, pl.ds(s * 16, 16)], i0)
                pltpu.sync_copy(idx_hbm.at[b, g, 1, pl.ds(s * 16, 16)], i1)
                pltpu.sync_copy(idx_hbm.at[b, g, 2, pl.ds(s * 16, 16)], i2)

                def far_j(j, _):
                    pbase = s * 2048 + j * 128
                    a0 = pltpu.async_copy(featg_hbm.at[i0.at[j]], fbuf, sem)
                    a1 = pltpu.async_copy(featg_hbm.at[i1.at[j]], r1, sem)
                    a2 = pltpu.async_copy(featg_hbm.at[i2.at[j]], r2, sem)
                    pltpu.sync_copy(wb_hbm.at[b, 0, pl.ds(pbase, 128)], w0)
                    pltpu.sync_copy(wb_hbm.at[b, 1, pl.ds(pbase, 128)], w1)
                    pltpu.sync_copy(wb_hbm.at[b, 2, pl.ds(pbase, 128)], w2)
                    a0.wait()
                    a1.wait()
                    a2.wait()

                    def interp(p, _):
                        obuf[p] = (fbuf[p] * w0[p] + r1[p] * w1[p]
                                   + r2[p] * w2[p])
                        return 0
                    lax.fori_loop(0, 128, interp, 0)
                    pltpu.sync_copy(obuf, slab.at[cbuf.at[j]], add=True)
                    return 0
                lax.fori_loop(0, 16, far_j, 0)
                plsc.subcore_barrier()
                copy_out(sums_hbm.at[b, g])
                plsc.subcore_barrier()

            zero_slab()
            plsc.subcore_barrier()
            pltpu.sync_copy(cells_hbm.at[b, c, pl.ds(s * 16, 16)], cbuf)

            def cnt_j(j, _):
                pltpu.sync_copy(onesb, slab.at[cbuf.at[j]], add=True)
                return 0
            lax.fori_loop(0, 16, cnt_j, 0)
            plsc.subcore_barrier()
            copy_out(cnt_hbm.at[b, c])
            plsc.subcore_barrier()

    return body(featg_flat, idxs, wb, cells_all)


def _cells(points):
    xi = jnp.clip(jnp.floor((points[..., 0] - _PC_MIN[0]) / _VOXEL[0]).astype(jnp.int32), 0, _NX - 1)
    yi = jnp.clip(jnp.floor((points[..., 1] - _PC_MIN[1]) / _VOXEL[1]).astype(jnp.int32), 0, _NY - 1)
    return yi * _NX + xi


def kernel(fv_features, points_img, proj_masks, points_img_far, proj_masks_far):
    b, c, h, w = fv_features.shape
    n = h * w
    feats = jnp.transpose(fv_features, (0, 2, 3, 1)).reshape(b, n, c)
    pts = jnp.transpose(points_img[:, :3], (0, 2, 3, 1)).reshape(b, n, 3)
    pts_far = jnp.transpose(points_img_far[:, :3], (0, 2, 3, 1)).reshape(b, n, 3)

    qn = (pts_far ** 2).sum(-1)
    kn = (pts ** 2).sum(-1)
    q_aug = jnp.concatenate([pts_far, qn[..., None]], axis=-1)
    k_aug = jnp.concatenate(
        [jnp.transpose(pts, (0, 2, 1)), kn[:, None, :]], axis=1)

    dist, idx = _three_nn_pallas(q_aug, k_aug)

    dist = jnp.maximum(dist, 0.0)
    recip = 1.0 / (dist + 1e-8)
    weight = recip / recip.sum(axis=-1, keepdims=True)

    cells_known = _cells(pts)
    cells_far = _cells(pts_far)

    featg_flat = jnp.transpose(feats.reshape(b, n, 4, 16),
                               (0, 2, 1, 3)).reshape(b * 4 * n, 16)
    offs = (jnp.arange(b, dtype=jnp.int32)[:, None] * 4
            + jnp.arange(4, dtype=jnp.int32)[None, :]) * n
    idx_t = jnp.transpose(idx, (0, 2, 1))
    idxs = (idx_t[:, None] + offs[:, :, None, None]).reshape(
        b, 4, 3, n // 128, 128)
    wb = jnp.broadcast_to(
        jnp.transpose(weight, (0, 2, 1))[..., None], (b, 3, n, 16))
    cells_all = jnp.stack([cells_known, cells_far], axis=1).reshape(
        b, 2, n // 128, 128)

    sums, cnt_part = _sc_scatter(featg_flat, idxs, wb, cells_all)

    cnt = cnt_part[:, 0, :, 0] + cnt_part[:, 1, :, 0]
    sums64 = jnp.transpose(sums, (0, 1, 3, 2)).reshape(b, c, _NY * _NX)
    bev = sums64 / jnp.maximum(cnt, 1.0)[:, None]
    return bev.reshape(b, c, _NY, _NX)
